# baseline (speedup 1.0000x reference)
import jax
import jax.numpy as jnp
from jax import lax
from jax.experimental import pallas as pl
from jax.experimental.pallas import tpu as pltpu
from jax.experimental.pallas import tpu_sc as plsc

B = 16384
D = 32

_info = plsc.get_sparse_core_info()
NC = _info.num_cores
NS = _info.num_subcores
NW = NC * NS
B_PER_W = B // NW


def _body(uid_hbm, mid_hbm, ut_hbm, mt_hbm, out_hbm,
          idxu_v, idxm_v, uT_v, mT_v, outv_v, sem_u, sem_m):
    wid = lax.axis_index("s") * NC + lax.axis_index("c")
    base = wid * B_PER_W

    pltpu.sync_copy(uid_hbm.at[pl.ds(base, B_PER_W)], idxu_v)
    pltpu.sync_copy(mid_hbm.at[pl.ds(base, B_PER_W)], idxm_v)

    def fire(c, _):
        s = c * 16
        vu = idxu_v[pl.ds(s, 16)]
        vm = idxm_v[pl.ds(s, 16)]
        for d in range(D):
            pltpu.async_copy(ut_hbm.at[d].at[vu],
                             uT_v.at[d, pl.ds(s, 16)], sem_u)
            pltpu.async_copy(mt_hbm.at[d].at[vm],
                             mT_v.at[d, pl.ds(s, 16)], sem_m)
        return 0

    lax.fori_loop(0, B_PER_W // 16, fire, 0)
    pltpu.make_async_copy(ut_hbm.at[pl.ds(0, D), pl.ds(0, B_PER_W)],
                          uT_v, sem_u).wait()
    pltpu.make_async_copy(mt_hbm.at[pl.ds(0, D), pl.ds(0, B_PER_W)],
                          mT_v, sem_m).wait()

    def blk(b, _):
        s = b * 16

        def dsum(d, acc):
            return acc + uT_v[d, pl.ds(s, 16)] * mT_v[d, pl.ds(s, 16)]

        outv_v[pl.ds(s, 16)] = lax.fori_loop(
            0, D, dsum, jnp.zeros((16,), jnp.float32))
        return 0

    lax.fori_loop(0, B_PER_W // 16, blk, 0)
    pltpu.sync_copy(outv_v, out_hbm.at[pl.ds(base, B_PER_W)])


@jax.jit
def _run(user_ids, movie_ids, ut, mt):
    mesh = plsc.VectorSubcoreMesh(core_axis_name="c", subcore_axis_name="s")
    k = pl.kernel(
        _body,
        mesh=mesh,
        out_type=jax.ShapeDtypeStruct((B,), jnp.float32),
        scratch_types=[
            pltpu.VMEM((B_PER_W,), jnp.int32),
            pltpu.VMEM((B_PER_W,), jnp.int32),
            pltpu.VMEM((D, B_PER_W), jnp.float32),
            pltpu.VMEM((D, B_PER_W), jnp.float32),
            pltpu.VMEM((B_PER_W,), jnp.float32),
            pltpu.SemaphoreType.DMA,
            pltpu.SemaphoreType.DMA,
        ],
        compiler_params=pltpu.CompilerParams(
            needs_layout_passes=False, use_tc_tiling_on_sc=False),
    )
    return k(user_ids, movie_ids, ut, mt)


def kernel(user_ids, movie_ids, user_table, movie_table):
    return _run(user_ids, movie_ids, user_table.T, movie_table.T)


# speedup vs baseline: 6.7771x; 6.7771x over previous
import jax
import jax.numpy as jnp
from jax import lax
from jax.experimental import pallas as pl
from jax.experimental.pallas import tpu as pltpu
from jax.experimental.pallas import tpu_sc as plsc

B = 16384
D = 32

_info = plsc.get_sparse_core_info()
NC = _info.num_cores
NS = _info.num_subcores
NW = NC * NS
B_PER_W = B // NW
CH = 32
NCHUNK = B_PER_W // CH


def _body(uid_hbm, mid_hbm, ut_hbm, mt_hbm, out_hbm,
          idxu_v, idxm_v, uwin_v, mwin_v, hb_v, outv_v, sem_u, sem_m):
    wid = lax.axis_index("s") * NC + lax.axis_index("c")
    base = wid * B_PER_W

    pltpu.sync_copy(uid_hbm.at[pl.ds(base, B_PER_W)], idxu_v)
    pltpu.sync_copy(mid_hbm.at[pl.ds(base, B_PER_W)], idxm_v)

    def chunk(h, _):
        hb = h * CH

        def fire(c, _):
            s = hb + c * 16
            vu = idxu_v[pl.ds(s, 16)]
            vm = idxm_v[pl.ds(s, 16)]
            for j in range(16):
                gu = pl.multiple_of((vu[j] >> 3) << 3, 8)
                gm = pl.multiple_of((vm[j] >> 3) << 3, 8)
                p = (c * 16 + j) * 8
                pltpu.async_copy(ut_hbm.at[pl.ds(gu, 8), :],
                                 uwin_v.at[pl.ds(p, 8), :], sem_u)
                pltpu.async_copy(mt_hbm.at[pl.ds(gm, 8), :],
                                 mwin_v.at[pl.ds(p, 8), :], sem_m)
            return 0

        lax.fori_loop(0, CH // 16, fire, 0)
        pltpu.make_async_copy(ut_hbm.at[pl.ds(0, CH * 8), :], uwin_v,
                              sem_u).wait()
        pltpu.make_async_copy(mt_hbm.at[pl.ds(0, CH * 8), :], mwin_v,
                              sem_m).wait()

        lanes16 = lax.iota(jnp.int32, 16) * 16

        def block(b, _):
            rbase = b * 16
            s = hb + rbase
            vu = idxu_v[pl.ds(s, 16)]
            vm = idxm_v[pl.ds(s, 16)]

            for r in range(16):
                pu = (rbase + r) * 8 + (vu[r] & 7)
                pm = (rbase + r) * 8 + (vm[r] & 7)
                p = (uwin_v[pu, pl.ds(0, 16)] * mwin_v[pm, pl.ds(0, 16)]
                     + uwin_v[pu, pl.ds(16, 16)] * mwin_v[pm, pl.ds(16, 16)])
                hb_v[pl.ds(r * 16, 16)] = p

            def tsum(j, acc):
                return acc + plsc.load_gather(hb_v, [lanes16 + j])

            outv_v[pl.ds(s, 16)] = lax.fori_loop(
                0, 16, tsum, jnp.zeros((16,), jnp.float32))
            return 0

        lax.fori_loop(0, CH // 16, block, 0)
        return 0

    lax.fori_loop(0, NCHUNK, chunk, 0)
    pltpu.sync_copy(outv_v, out_hbm.at[pl.ds(base, B_PER_W)])


@jax.jit
def _run(user_ids, movie_ids, ut, mt):
    mesh = plsc.VectorSubcoreMesh(core_axis_name="c", subcore_axis_name="s")
    k = pl.kernel(
        _body,
        mesh=mesh,
        out_type=jax.ShapeDtypeStruct((B,), jnp.float32),
        scratch_types=[
            pltpu.VMEM((B_PER_W,), jnp.int32),
            pltpu.VMEM((B_PER_W,), jnp.int32),
            pltpu.VMEM((CH * 8, D), jnp.float32),
            pltpu.VMEM((CH * 8, D), jnp.float32),
            pltpu.VMEM((16 * 16,), jnp.float32),
            pltpu.VMEM((B_PER_W,), jnp.float32),
            pltpu.SemaphoreType.DMA,
            pltpu.SemaphoreType.DMA,
        ],
        compiler_params=pltpu.CompilerParams(
            needs_layout_passes=False, use_tc_tiling_on_sc=True),
    )
    return k(user_ids, movie_ids, ut, mt)


def kernel(user_ids, movie_ids, user_table, movie_table):
    return _run(user_ids, movie_ids, user_table, movie_table)
